# baseline (speedup 1.0000x reference)
import functools

import jax
import jax.numpy as jnp
from jax import lax
from jax.experimental import pallas as pl
from jax.experimental.pallas import tpu as pltpu
from jax.experimental.pallas import tpu_sc as plsc

_D = 1000
_NW = 32
_NBUF = 2


def _make_gather(batch, seq):
  batches_per_w = batch // _NW
  mesh = plsc.VectorSubcoreMesh(core_axis_name="c", subcore_axis_name="s")

  @functools.partial(
      pl.kernel,
      out_type=jax.ShapeDtypeStruct((batch, seq, _D), jnp.float32),
      mesh=mesh,
      compiler_params=pltpu.CompilerParams(use_tc_tiling_on_sc=False),
      scratch_types=[
          pltpu.VMEM((batches_per_w, seq), jnp.int32),
          pltpu.VMEM((_NBUF, seq, _D), jnp.float32),
      ] + [pltpu.SemaphoreType.DMA] * (2 * _NBUF),
  )
  def gather_kernel(idx_hbm, table_hbm, out_hbm, idx_v, rows_v, *sems):
    wid = lax.axis_index("s") * 2 + lax.axis_index("c")
    base = wid * batches_per_w
    pltpu.sync_copy(idx_hbm.at[wid], idx_v)

    sems_g = sems[:_NBUF]
    sems_o = sems[_NBUF:]

    def start_gather(j, b):
      pltpu.async_copy(table_hbm.at[idx_v.at[j]], rows_v.at[b], sems_g[b])

    def wait_gather(j, b):
      pltpu.make_async_copy(table_hbm.at[idx_v.at[j]], rows_v.at[b],
                            sems_g[b]).wait()

    def start_write(j, b):
      pltpu.async_copy(rows_v.at[b], out_hbm.at[base + j], sems_o[b])

    def wait_write(j, b):
      pltpu.make_async_copy(rows_v.at[b], out_hbm.at[base + j],
                            sems_o[b]).wait()

    for b in range(_NBUF):
      start_gather(b, b)

    @pl.loop(0, batches_per_w - _NBUF, step=_NBUF)
    def _(j0):
      for b in range(_NBUF):
        j = j0 + b
        wait_gather(j, b)
        start_write(j, b)
        wait_write(j, b)
        start_gather(j + _NBUF, b)

    for b in range(_NBUF):
      j = batches_per_w - _NBUF + b
      wait_gather(j, b)
      start_write(j, b)
      wait_write(j, b)

  return gather_kernel


@jax.jit
def kernel(indices, embedding):
  batch, seq = indices.shape
  idx = indices.astype(jnp.int32).reshape(_NW, batch // _NW, seq)
  return _make_gather(batch, seq)(idx, embedding)
